# Initial kernel scaffold; baseline (speedup 1.0000x reference)
import jax
import jax.numpy as jnp
from jax.experimental import pallas as pl


def kernel(x, W_lin, b_lin, emb):
    raise NotImplementedError("write your pallas kernel here")


# speedup vs baseline: 1.1861x; 1.1861x over previous
import functools

import jax
import jax.numpy as jnp
from jax.experimental import pallas as pl
from jax.experimental.pallas import tpu as pltpu

_B, _C, _H, _W = 16, 64, 32, 32
_HW = _H * _W
_K = 1024
_N = _B * _HW
_COMMIT = 0.25


def _vq_body(x_ref, w_ref, b_ref, emb_ref,
             idx_ref, q_ref, loss_ref, perp_ref,
             counts_acc, dsum_acc):
    b = pl.program_id(0)
    x = x_ref[0]
    w = w_ref[...]
    emb = emb_ref[...]

    z = jax.lax.dot_general(w, x, (((1,), (0,)), ((), ()))) + b_ref[...]

    zsq = jnp.sum(z * z, axis=0, keepdims=True)
    esq = jnp.sum(emb * emb, axis=1, keepdims=True)
    s = jax.lax.dot_general(emb, z, (((1,), (0,)), ((), ())))
    dist = (zsq + esq) - 2.0 * s

    m = jnp.min(dist, axis=0, keepdims=True)
    kiota = jax.lax.broadcasted_iota(jnp.int32, (_K, _HW), 0)
    idx = jnp.min(jnp.where(dist == m, kiota, _K), axis=0, keepdims=True)
    idx_ref[0] = idx

    onehot = (kiota == idx).astype(jnp.float32)
    q = jax.lax.dot_general(emb, onehot, (((0,), (0,)), ((), ())))
    q_ref[0] = q

    cnt_b = jnp.sum(onehot, axis=1, keepdims=True)
    dsum_b = jnp.sum(m)

    @pl.when(b == 0)
    def _init():
        counts_acc[...] = cnt_b
        dsum_acc[0] = dsum_b

    @pl.when(b > 0)
    def _acc():
        counts_acc[...] = counts_acc[...] + cnt_b
        dsum_acc[0] = dsum_acc[0] + dsum_b

    @pl.when(b == _B - 1)
    def _fin():
        loss = (1.0 + _COMMIT) * dsum_acc[0] / float(_N * _C)
        loss_ref[...] = jnp.broadcast_to(loss, (1, 1))
        p = counts_acc[...] * (1.0 / float(_N))
        perp = jnp.exp(-jnp.sum(p * jnp.log(p + 1e-10)))
        perp_ref[...] = jnp.broadcast_to(perp, (1, 1))


@functools.partial(jax.jit, static_argnums=())
def kernel(x, W_lin, b_lin, emb):
    xr = x.reshape(_B, _C, _HW)
    br = b_lin.reshape(_C, 1)
    idx, q, loss, perp = pl.pallas_call(
        _vq_body,
        grid=(_B,),
        in_specs=[
            pl.BlockSpec((1, _C, _HW), lambda b: (b, 0, 0)),
            pl.BlockSpec((_C, _C), lambda b: (0, 0)),
            pl.BlockSpec((_C, 1), lambda b: (0, 0)),
            pl.BlockSpec((_K, _C), lambda b: (0, 0)),
        ],
        out_specs=[
            pl.BlockSpec((1, 1, _HW), lambda b: (b, 0, 0)),
            pl.BlockSpec((1, _C, _HW), lambda b: (b, 0, 0)),
            pl.BlockSpec((1, 1), lambda b: (0, 0)),
            pl.BlockSpec((1, 1), lambda b: (0, 0)),
        ],
        out_shape=[
            jax.ShapeDtypeStruct((_B, 1, _HW), jnp.int32),
            jax.ShapeDtypeStruct((_B, _C, _HW), jnp.float32),
            jax.ShapeDtypeStruct((1, 1), jnp.float32),
            jax.ShapeDtypeStruct((1, 1), jnp.float32),
        ],
        scratch_shapes=[
            pltpu.VMEM((_K, 1), jnp.float32),
            pltpu.SMEM((1,), jnp.float32),
        ],
    )(xr, W_lin, br, emb)
    return (loss[0, 0],
            q.reshape(_B, _C, _H, _W),
            perp[0, 0],
            idx.reshape(_N)[:, None])
